# baseline (speedup 1.0000x reference)
import math

import jax
import jax.numpy as jnp
from jax.experimental import pallas as pl
from jax.experimental.pallas import tpu as pltpu


def _pe_add_kernel(scale, x_ref, pe_ref, o_ref):
    o_ref[...] = x_ref[...] * scale + pe_ref[...]


def kernel(x, pe):
    B, T, H, W, S = x.shape
    HW = H * W
    scale = math.sqrt(S)

    x4 = x.reshape(B, T, HW, S)
    pe3 = pe[:T].reshape(T, 1, S)

    TB = T // 2
    out = pl.pallas_call(
        lambda x_ref, pe_ref, o_ref: _pe_add_kernel(scale, x_ref, pe_ref, o_ref),
        grid=(B, 2),
        in_specs=[
            pl.BlockSpec((1, TB, HW, S), lambda b, th: (b, th, 0, 0)),
            pl.BlockSpec((TB, 1, S), lambda b, th: (th, 0, 0)),
        ],
        out_specs=pl.BlockSpec((1, TB, HW, S), lambda b, th: (b, th, 0, 0)),
        out_shape=jax.ShapeDtypeStruct((B, T, HW, S), x.dtype),
        compiler_params=pltpu.CompilerParams(
            dimension_semantics=("parallel", "parallel"),
        ),
    )(x4, pe3)

    return out.reshape(B, T, H, W, S)


# speedup vs baseline: 1.0085x; 1.0085x over previous
import math

import jax
import jax.numpy as jnp
from jax.experimental import pallas as pl
from jax.experimental.pallas import tpu as pltpu


def _pe_add_kernel(scale, x_ref, pe_ref, o_ref):
    o_ref[...] = x_ref[...] * scale + pe_ref[...]


def kernel(x, pe):
    B, T, H, W, S = x.shape
    HW = H * W
    scale = math.sqrt(S)

    x3 = x.reshape(B * T, HW, S)
    pe3 = pe[:T].reshape(T, 1, S)

    out = pl.pallas_call(
        lambda x_ref, pe_ref, o_ref: _pe_add_kernel(scale, x_ref, pe_ref, o_ref),
        grid=(B,),
        in_specs=[
            pl.BlockSpec((T, HW, S), lambda b: (b, 0, 0)),
            pl.BlockSpec((T, 1, S), lambda b: (0, 0, 0)),
        ],
        out_specs=pl.BlockSpec((T, HW, S), lambda b: (b, 0, 0)),
        out_shape=jax.ShapeDtypeStruct((B * T, HW, S), x.dtype),
        compiler_params=pltpu.CompilerParams(
            dimension_semantics=("parallel",),
        ),
    )(x3, pe3)

    return out.reshape(B, T, H, W, S)
